# baseline (speedup 1.0000x reference)
import jax
import jax.numpy as jnp
from jax import lax
from jax.experimental import pallas as pl
from jax.experimental.pallas import tpu as pltpu
from jax.experimental.pallas import tpu_sc as plsc

K = 8192
D = 64
N = 16384
BETA = 0.25

TM = 512
TK = 1024
_LANES = 128
_NSL = TK // _LANES
_NB = N // TM
_CPB = K // _NB


def _init_body(z_ref, cb_ref, pw_ref, pb_ref, qc_ref, qcm2_ref, qcsqb_ref, zsqb_ref):
    qc = lax.dot_general(
        cb_ref[...], pw_ref[...],
        dimension_numbers=(((1,), (1,)), ((), ())),
        preferred_element_type=jnp.float32,
        precision=lax.Precision.DEFAULT,
    ) + pb_ref[...]
    qc_ref[...] = qc
    qcm2_ref[...] = -2.0 * qc
    qcsq = jnp.sum(qc * qc, axis=1)
    qcsqb_ref[...] = jnp.broadcast_to(
        qcsq.reshape(_CPB // _LANES, 1, _LANES), (_CPB // _LANES, 8, _LANES))
    zb = z_ref[...]
    zsq = jnp.sum(zb * zb, axis=1)
    zsqb_ref[...] = jnp.broadcast_to(zsq[:, None], (TM, _LANES))


def _init(zf, codebook, proj_w, proj_b):
    return pl.pallas_call(
        _init_body,
        grid=(_NB,),
        in_specs=[
            pl.BlockSpec((TM, D), lambda i: (i, 0)),
            pl.BlockSpec((_CPB, D), lambda i: (i, 0)),
            pl.BlockSpec((D, D), lambda i: (0, 0)),
            pl.BlockSpec((1, D), lambda i: (0, 0)),
        ],
        out_specs=(
            pl.BlockSpec((_CPB, D), lambda i: (i, 0)),
            pl.BlockSpec((_CPB, D), lambda i: (i, 0)),
            pl.BlockSpec((_CPB // _LANES, 8, _LANES), lambda i: (i, 0, 0)),
            pl.BlockSpec((TM, _LANES), lambda i: (i, 0)),
        ),
        out_shape=(
            jax.ShapeDtypeStruct((K, D), jnp.float32),
            jax.ShapeDtypeStruct((K, D), jnp.float32),
            jax.ShapeDtypeStruct((K // _LANES, 8, _LANES), jnp.float32),
            jax.ShapeDtypeStruct((N, _LANES), jnp.float32),
        ),
    )(zf, codebook, proj_w, proj_b.reshape(1, D))


def _scan_body(z_ref, qcm2_ref, qcsqb_ref, zsqb_ref, rmin_ref, rcid_ref):
    j = pl.program_id(1)
    s2 = lax.dot_general(
        z_ref[...], qcm2_ref[...],
        dimension_numbers=(((1,), (1,)), ((), ())),
        preferred_element_type=jnp.float32,
        precision=lax.Precision.DEFAULT,
    )
    zs = zsqb_ref[...]
    ds = []
    for k in range(_NSL):
        qk = jnp.broadcast_to(qcsqb_ref[0, k][None], (TM // 8, 8, _LANES)).reshape(TM, _LANES)
        ds.append(((zs + qk) + s2[:, k * _LANES:(k + 1) * _LANES], k))
    while len(ds) > 1:
        nxt = []
        for a in range(0, len(ds), 2):
            (d0, c0), (d1, c1) = ds[a], ds[a + 1]
            lt = d1 < d0
            if isinstance(c0, int):
                cw = jnp.where(lt, jnp.int32(c1), jnp.int32(c0))
            else:
                cw = jnp.where(lt, c1, c0)
            nxt.append((jnp.minimum(d0, d1), cw))
        ds = nxt
    dw, cw = ds[0]
    first = j == 0
    rm = jnp.where(first, jnp.inf, rmin_ref[...])
    rc = jnp.where(first, 0, rcid_ref[...])
    lt = dw < rm
    rmin_ref[...] = jnp.minimum(rm, dw)
    rcid_ref[...] = jnp.where(lt, cw + j * _NSL, rc)


def _reduce_body(rmin_ref, rcid_ref, out_ref):
    rm = rmin_ref[...]
    m = jnp.min(rm, axis=1)
    lane = lax.broadcasted_iota(jnp.int32, (TM, _LANES), 1)
    glob = rcid_ref[...] * _LANES + lane
    masked = jnp.where(rm == m[:, None], glob, jnp.int32(2**30))
    out_ref[...] = jnp.min(masked, axis=1)


def _argmin(zf, qcm2, qcsqb, zsqb):
    rm, rc = pl.pallas_call(
        _scan_body,
        grid=(_NB, K // TK),
        in_specs=[
            pl.BlockSpec((TM, D), lambda i, j: (i, 0)),
            pl.BlockSpec((TK, D), lambda i, j: (j, 0)),
            pl.BlockSpec((1, _NSL, 8, _LANES), lambda i, j: (j, 0, 0, 0)),
            pl.BlockSpec((TM, _LANES), lambda i, j: (i, 0)),
        ],
        out_specs=(
            pl.BlockSpec((TM, _LANES), lambda i, j: (i, 0)),
            pl.BlockSpec((TM, _LANES), lambda i, j: (i, 0)),
        ),
        out_shape=(
            jax.ShapeDtypeStruct((N, _LANES), jnp.float32),
            jax.ShapeDtypeStruct((N, _LANES), jnp.int32),
        ),
        compiler_params=pltpu.CompilerParams(
            dimension_semantics=("arbitrary", "arbitrary"),
        ),
    )(zf, qcm2, qcsqb, zsqb)
    return pl.pallas_call(
        _reduce_body,
        grid=(_NB,),
        in_specs=[
            pl.BlockSpec((TM, _LANES), lambda i: (i, 0)),
            pl.BlockSpec((TM, _LANES), lambda i: (i, 0)),
        ],
        out_specs=pl.BlockSpec((TM,), lambda i: (i,)),
        out_shape=jax.ShapeDtypeStruct((N,), jnp.int32),
    )(rm, rc)


_SC_NC = 2
_SC_NS = 16
_BPW = N // (_SC_NC * _SC_NS)
_KPS = K // _SC_NS


def _sc_body(qc_hbm, idx_hbm, zq_hbm, cnt_hbm,
             idx_v, rows_v, stage_v, ones_v, cnt_sh, sem):
    c = lax.axis_index("c")
    s = lax.axis_index("s")
    wid = c * _SC_NS + s
    base = wid * _BPW

    pltpu.sync_copy(idx_hbm.at[pl.ds(base, _BPW)], idx_v)
    pltpu.async_copy(qc_hbm.at[idx_v], rows_v, sem).wait()
    pltpu.sync_copy(rows_v, zq_hbm.at[pl.ds(base, _BPW)])

    def fill(i, _):
        ones_v[pl.ds(i * 16, 16)] = jnp.full((16,), 1, jnp.int32)
        stage_v[pl.ds(i * 16, 16)] = jnp.full((16,), 0, jnp.int32)
        return 0

    lax.fori_loop(0, _BPW // 16, fill, 0)

    pltpu.sync_copy(stage_v, cnt_sh.at[pl.ds(s * _KPS, _KPS)])
    plsc.subcore_barrier()
    pltpu.sync_copy(ones_v, cnt_sh.at[idx_v], add=True)
    plsc.subcore_barrier()

    pltpu.sync_copy(cnt_sh.at[pl.ds(s * _KPS, _KPS)], stage_v)
    pltpu.sync_copy(stage_v, cnt_hbm.at[c, pl.ds(s * _KPS, _KPS)])


def _gather_counts(qc, idx):
    mesh = plsc.VectorSubcoreMesh(core_axis_name="c", subcore_axis_name="s")
    f = pl.kernel(
        _sc_body,
        out_type=(
            jax.ShapeDtypeStruct((N, D), jnp.float32),
            jax.ShapeDtypeStruct((_SC_NC, K), jnp.int32),
        ),
        mesh=mesh,
        scratch_types=[
            pltpu.VMEM((_BPW,), jnp.int32),
            pltpu.VMEM((_BPW, D), jnp.float32),
            pltpu.VMEM((_KPS,), jnp.int32),
            pltpu.VMEM((_BPW,), jnp.int32),
            pltpu.VMEM_SHARED((K,), jnp.int32),
            pltpu.SemaphoreType.DMA,
        ],
        compiler_params=pltpu.CompilerParams(use_tc_tiling_on_sc=False),
    )
    return f(qc, idx)


def _loss_body(zf_ref, zq_ref, cnt_ref, loss_ref, perp_ref):
    diff = zq_ref[...] - zf_ref[...]
    sq = jnp.sum(diff * diff)
    loss_ref[0, 0] = (1.0 + BETA) * sq / jnp.float32(N * D)
    counts = cnt_ref[0:K] + cnt_ref[K:2 * K]
    e = counts.astype(jnp.float32) * jnp.float32(1.0 / N)
    ent = jnp.sum(e * jnp.log(e + 1e-8))
    perp_ref[0, 0] = jnp.exp(-ent)


def _losses(zf, zq, cnt):
    return pl.pallas_call(
        _loss_body,
        in_specs=[
            pl.BlockSpec(memory_space=pltpu.VMEM),
            pl.BlockSpec(memory_space=pltpu.VMEM),
            pl.BlockSpec(memory_space=pltpu.VMEM),
        ],
        out_specs=(
            pl.BlockSpec(memory_space=pltpu.SMEM),
            pl.BlockSpec(memory_space=pltpu.SMEM),
        ),
        out_shape=(
            jax.ShapeDtypeStruct((1, 1), jnp.float32),
            jax.ShapeDtypeStruct((1, 1), jnp.float32),
        ),
    )(zf, zq, cnt)


def kernel(z, codebook, proj_w, proj_b):
    zf = z.reshape(-1, D)
    qc, qcm2, qcsqb, zsqb = _init(zf, codebook, proj_w, proj_b)
    idx = _argmin(zf, qcm2, qcsqb.reshape(K // TK, _NSL, 8, _LANES), zsqb)
    zq, cnt = _gather_counts(qc, idx)
    loss, perp = _losses(zf, zq, cnt.reshape(-1))
    return zq.reshape(z.shape), loss[0, 0], perp[0, 0]


# speedup vs baseline: 1.7958x; 1.7958x over previous
import jax
import jax.numpy as jnp
from jax import lax
from jax.experimental import pallas as pl
from jax.experimental.pallas import tpu as pltpu
from jax.experimental.pallas import tpu_sc as plsc

K = 8192
D = 64
N = 16384
BETA = 0.25

TM = 1024
TK = 8192
_LANES = 128
_NSL = TK // _LANES
_NB = N // TM
_CPB = K // _NB


def _scan_body(z_ref, cb_ref, pw_ref, pb_ref, idx_ref, loss_ref, qc_ref,
               qcm2_ref, qcsqb_ref, acc_ref):
    i = pl.program_id(0)
    zb = z_ref[...]

    @pl.when(i == 0)
    def _():
        qc = lax.dot_general(
            cb_ref[...], pw_ref[...],
            dimension_numbers=(((1,), (1,)), ((), ())),
            preferred_element_type=jnp.float32,
            precision=lax.Precision.DEFAULT,
        ) + pb_ref[...]
        qc_ref[...] = qc
        qcm2_ref[...] = -2.0 * qc
        qcsq = jnp.sum(qc * qc, axis=1)
        qcsqb_ref[...] = jnp.broadcast_to(
            qcsq.reshape(K // _LANES, 1, _LANES), (K // _LANES, 8, _LANES))

    zsq = jnp.sum(zb * zb, axis=1)
    zs = jnp.broadcast_to(zsq[:, None], (TM, _LANES))
    s2 = lax.dot_general(
        zb, qcm2_ref[...],
        dimension_numbers=(((1,), (1,)), ((), ())),
        preferred_element_type=jnp.float32,
        precision=lax.Precision.DEFAULT,
    )
    ds = []
    for k in range(_NSL):
        qk = jnp.broadcast_to(qcsqb_ref[k][None], (TM // 8, 8, _LANES)).reshape(TM, _LANES)
        ds.append(((zs + qk) + s2[:, k * _LANES:(k + 1) * _LANES], k))
    while len(ds) > 1:
        nxt = []
        for a in range(0, len(ds), 2):
            (d0, c0), (d1, c1) = ds[a], ds[a + 1]
            lt = d1 < d0
            if isinstance(c0, int):
                cw = jnp.where(lt, jnp.int32(c1), jnp.int32(c0))
            else:
                cw = jnp.where(lt, c1, c0)
            nxt.append((jnp.minimum(d0, d1), cw))
        ds = nxt
    dw, cw = ds[0]
    m = jnp.min(dw, axis=1)
    lane = lax.broadcasted_iota(jnp.int32, (TM, _LANES), 1)
    glob = cw * _LANES + lane
    masked = jnp.where(dw == m[:, None], glob, jnp.int32(2**30))
    idx_ref[...] = jnp.min(masked, axis=1)
    blk = jnp.sum(m)

    @pl.when(i == 0)
    def _():
        acc_ref[0] = blk

    @pl.when(i > 0)
    def _():
        acc_ref[0] = acc_ref[0] + blk

    @pl.when(i == _NB - 1)
    def _():
        loss_ref[0, 0] = (1.0 + BETA) * acc_ref[0] / jnp.float32(N * D)


def _argmin(zf, codebook, proj_w, proj_b):
    return pl.pallas_call(
        _scan_body,
        grid=(_NB,),
        in_specs=[
            pl.BlockSpec((TM, D), lambda i: (i, 0)),
            pl.BlockSpec((K, D), lambda i: (0, 0)),
            pl.BlockSpec((D, D), lambda i: (0, 0)),
            pl.BlockSpec((1, D), lambda i: (0, 0)),
        ],
        out_specs=(
            pl.BlockSpec((TM,), lambda i: (i,)),
            pl.BlockSpec((1, 1), lambda i: (0, 0), memory_space=pltpu.SMEM),
            pl.BlockSpec((K, D), lambda i: (0, 0)),
        ),
        out_shape=(
            jax.ShapeDtypeStruct((N,), jnp.int32),
            jax.ShapeDtypeStruct((1, 1), jnp.float32),
            jax.ShapeDtypeStruct((K, D), jnp.float32),
        ),
        scratch_shapes=[
            pltpu.VMEM((K, D), jnp.float32),
            pltpu.VMEM((K // _LANES, 8, _LANES), jnp.float32),
            pltpu.SMEM((1,), jnp.float32),
        ],
        compiler_params=pltpu.CompilerParams(
            dimension_semantics=("arbitrary",),
        ),
    )(zf, codebook, proj_w, proj_b.reshape(1, D))


_SC_NC = 2
_SC_NS = 16
_BPW = N // (_SC_NC * _SC_NS)
_KPS = K // _SC_NS


def _sc_body(qc_hbm, idx_hbm, zq_hbm, cnt_hbm,
             idx_v, rows_v, stage_v, ones_v, cnt_sh, sem):
    c = lax.axis_index("c")
    s = lax.axis_index("s")
    wid = c * _SC_NS + s
    base = wid * _BPW

    pltpu.sync_copy(idx_hbm.at[pl.ds(base, _BPW)], idx_v)
    pltpu.async_copy(qc_hbm.at[idx_v], rows_v, sem).wait()
    pltpu.sync_copy(rows_v, zq_hbm.at[pl.ds(base, _BPW)])

    def fill(i, _):
        ones_v[pl.ds(i * 16, 16)] = jnp.full((16,), 1, jnp.int32)
        stage_v[pl.ds(i * 16, 16)] = jnp.full((16,), 0, jnp.int32)
        return 0

    lax.fori_loop(0, _BPW // 16, fill, 0)

    pltpu.sync_copy(stage_v, cnt_sh.at[pl.ds(s * _KPS, _KPS)])
    plsc.subcore_barrier()
    pltpu.sync_copy(ones_v, cnt_sh.at[idx_v], add=True)
    plsc.subcore_barrier()

    pltpu.sync_copy(cnt_sh.at[pl.ds(s * _KPS, _KPS)], stage_v)
    pltpu.sync_copy(stage_v, cnt_hbm.at[c, pl.ds(s * _KPS, _KPS)])


def _gather_counts(qc, idx):
    mesh = plsc.VectorSubcoreMesh(core_axis_name="c", subcore_axis_name="s")
    f = pl.kernel(
        _sc_body,
        out_type=(
            jax.ShapeDtypeStruct((N, D), jnp.float32),
            jax.ShapeDtypeStruct((_SC_NC, K), jnp.int32),
        ),
        mesh=mesh,
        scratch_types=[
            pltpu.VMEM((_BPW,), jnp.int32),
            pltpu.VMEM((_BPW, D), jnp.float32),
            pltpu.VMEM((_KPS,), jnp.int32),
            pltpu.VMEM((_BPW,), jnp.int32),
            pltpu.VMEM_SHARED((K,), jnp.int32),
            pltpu.SemaphoreType.DMA,
        ],
        compiler_params=pltpu.CompilerParams(use_tc_tiling_on_sc=False),
    )
    return f(qc, idx)


def _perp_body(cnt_ref, perp_ref):
    counts = cnt_ref[0:K] + cnt_ref[K:2 * K]
    e = counts.astype(jnp.float32) * jnp.float32(1.0 / N)
    ent = jnp.sum(e * jnp.log(e + 1e-8))
    perp_ref[0, 0] = jnp.exp(-ent)


def _perplexity(cnt):
    return pl.pallas_call(
        _perp_body,
        in_specs=[pl.BlockSpec(memory_space=pltpu.VMEM)],
        out_specs=pl.BlockSpec(memory_space=pltpu.SMEM),
        out_shape=jax.ShapeDtypeStruct((1, 1), jnp.float32),
    )(cnt)


def kernel(z, codebook, proj_w, proj_b):
    zf = z.reshape(-1, D)
    idx, loss, qc = _argmin(zf, codebook, proj_w, proj_b)
    zq, cnt = _gather_counts(qc, idx)
    perp = _perplexity(cnt.reshape(-1))
    return zq.reshape(z.shape), loss[0, 0], perp[0, 0]
